# baseline (speedup 1.0000x reference)
import functools
import jax
import jax.numpy as jnp
import numpy as np
from jax.experimental import pallas as pl

_N = 50000
_E = 800000
_D = 100
_DP = 128
_DE = 16
_L = 4

_NODE_BLK = 1000
_EDGE_BLK = 4000
_FLAT_BLK = 1250


def _mm_kernel(x_ref, w_ref, o_ref):
    o_ref[...] = jnp.dot(x_ref[...], w_ref[...],
                         preferred_element_type=jnp.float32)


def _matmul(x, w):
    n = x.shape[0]
    blk = _NODE_BLK
    return pl.pallas_call(
        _mm_kernel,
        grid=(n // blk,),
        in_specs=[pl.BlockSpec((blk, _DP), lambda i: (i, 0)),
                  pl.BlockSpec((_DP, _DP), lambda i: (0, 0))],
        out_specs=pl.BlockSpec((blk, _DP), lambda i: (i, 0)),
        out_shape=jax.ShapeDtypeStruct((n, _DP), jnp.float32),
    )(x, w)


def _msg_kernel(g_ref, ea_ref, we_ref, b_ref, o_ref):
    m = g_ref[...] + jnp.dot(ea_ref[...], we_ref[...],
                             preferred_element_type=jnp.float32) + b_ref[...]
    o_ref[...] = jnp.maximum(m, 0.0)


def _messages(g, ea, we, b):
    blk = _EDGE_BLK
    return pl.pallas_call(
        _msg_kernel,
        grid=(_E // blk,),
        in_specs=[pl.BlockSpec((blk, _DP), lambda i: (i, 0)),
                  pl.BlockSpec((blk, _DE), lambda i: (i, 0)),
                  pl.BlockSpec((_DE, _DP), lambda i: (0, 0)),
                  pl.BlockSpec((1, _DP), lambda i: (0, 0))],
        out_specs=pl.BlockSpec((blk, _DP), lambda i: (i, 0)),
        out_shape=jax.ShapeDtypeStruct((_E, _DP), jnp.float32),
    )(g, ea, we, b)


def _update_kernel(agg_ref, h_ref, w2_ref, ws_ref, b_ref, wn_ref,
                   o_ref, hw_ref):
    hn = (jnp.dot(agg_ref[...], w2_ref[...],
                  preferred_element_type=jnp.float32)
          + jnp.dot(h_ref[...], ws_ref[...],
                    preferred_element_type=jnp.float32)
          + b_ref[...])
    hn = jnp.maximum(hn, 0.0)
    o_ref[...] = hn
    hw_ref[...] = jnp.dot(hn, wn_ref[...],
                          preferred_element_type=jnp.float32)


def _update(agg, h, w2, ws, b, wnext):
    blk = _NODE_BLK
    return pl.pallas_call(
        _update_kernel,
        grid=(_N // blk,),
        in_specs=[pl.BlockSpec((blk, _DP), lambda i: (i, 0)),
                  pl.BlockSpec((blk, _DP), lambda i: (i, 0)),
                  pl.BlockSpec((_DP, _DP), lambda i: (0, 0)),
                  pl.BlockSpec((_DP, _DP), lambda i: (0, 0)),
                  pl.BlockSpec((1, _DP), lambda i: (0, 0)),
                  pl.BlockSpec((_DP, _DP), lambda i: (0, 0))],
        out_specs=[pl.BlockSpec((blk, _DP), lambda i: (i, 0)),
                   pl.BlockSpec((blk, _DP), lambda i: (i, 0))],
        out_shape=[jax.ShapeDtypeStruct((_N, _DP), jnp.float32),
                   jax.ShapeDtypeStruct((_N, _DP), jnp.float32)],
    )(agg, h, w2, ws, b, wnext)


def _dot_kernel(a_ref, b_ref, o_ref):
    o_ref[...] = jnp.sum(a_ref[...] * b_ref[...], axis=1, keepdims=True) * 0.1


def _edge_dots(hs, hd):
    blk = _EDGE_BLK
    return pl.pallas_call(
        _dot_kernel,
        grid=(_E // blk,),
        in_specs=[pl.BlockSpec((blk, _DP), lambda i: (i, 0)),
                  pl.BlockSpec((blk, _DP), lambda i: (i, 0))],
        out_specs=pl.BlockSpec((blk, 1), lambda i: (i, 0)),
        out_shape=jax.ShapeDtypeStruct((_E, 1), jnp.float32),
    )(hs, hd)


def _exp_kernel(s_ref, m_ref, o_ref):
    o_ref[...] = jnp.exp(s_ref[...] - m_ref[...])


def _div_kernel(e_ref, d_ref, o_ref):
    o_ref[...] = e_ref[...] / d_ref[...]


def _flat_ew(body, a, b):
    rows = _E // 128
    return pl.pallas_call(
        body,
        out_shape=jax.ShapeDtypeStruct((rows, 128), jnp.float32),
    )(a, b)


def kernel(x, edge_index, edge_attributes, W1, We, b1, W2, Wself, b2):
    src = edge_index[0]
    dst = edge_index[1]

    pad = _DP - _D
    xp = jnp.pad(x, ((0, 0), (0, pad)))
    W1p = jnp.pad(W1, ((0, 0), (0, pad), (0, pad)))
    Wep = jnp.pad(We, ((0, 0), (0, 0), (0, pad)))
    b1p = jnp.pad(b1, ((0, 0), (0, pad))).reshape(_L, 1, _DP)
    W2p = jnp.pad(W2, ((0, 0), (0, pad), (0, pad)))
    Wsp = jnp.pad(Wself, ((0, 0), (0, pad), (0, pad)))
    b2p = jnp.pad(b2, ((0, 0), (0, pad))).reshape(_L, 1, _DP)

    h = xp
    hw = _matmul(h, W1p[0])
    for i in range(_L):
        g = jnp.take(hw, src, axis=0)
        m = _messages(g, edge_attributes, Wep[i], b1p[i])
        agg = jax.ops.segment_sum(m, dst, num_segments=_N)
        wnext = W1p[i + 1] if i + 1 < _L else W1p[0]
        h, hw = _update(agg, h, W2p[i], Wsp[i], b2p[i], wnext)

    hs = jnp.take(h, src, axis=0)
    hd = jnp.take(h, dst, axis=0)
    s = _edge_dots(hs, hd).reshape(_E // 128, 128)

    sf = s.reshape(_E)
    smax = jax.ops.segment_max(sf, src, num_segments=_N)
    smax_g = jnp.take(smax, src, axis=0).reshape(_E // 128, 128)
    ex = _flat_ew(_exp_kernel, s, smax_g)
    denom = jax.ops.segment_sum(ex.reshape(_E), src, num_segments=_N)
    denom_g = jnp.take(denom, src, axis=0).reshape(_E // 128, 128)
    pi = _flat_ew(_div_kernel, ex, denom_g)
    return pi.reshape(_E)


# speedup vs baseline: 1.3663x; 1.3663x over previous
import functools
import jax
import jax.numpy as jnp
import numpy as np
from jax import lax
from jax.experimental import pallas as pl
from jax.experimental.pallas import tpu as pltpu
from jax.experimental.pallas import tpu_sc as plsc

_N = 50000
_E = 800000
_D = 100
_DP = 128
_DE = 16
_L = 4

_NODE_BLK = 1000
_EDGE_BLK = 4000
_FLAT_BLK = 1250


def _mm_kernel(x_ref, w_ref, o_ref):
    o_ref[...] = jnp.dot(x_ref[...], w_ref[...],
                         preferred_element_type=jnp.float32)


def _matmul(x, w):
    n = x.shape[0]
    blk = _NODE_BLK
    return pl.pallas_call(
        _mm_kernel,
        grid=(n // blk,),
        in_specs=[pl.BlockSpec((blk, _DP), lambda i: (i, 0)),
                  pl.BlockSpec((_DP, _DP), lambda i: (0, 0))],
        out_specs=pl.BlockSpec((blk, _DP), lambda i: (i, 0)),
        out_shape=jax.ShapeDtypeStruct((n, _DP), jnp.float32),
    )(x, w)


def _msg_kernel(g_ref, ea_ref, we_ref, b_ref, o_ref):
    m = g_ref[...] + jnp.dot(ea_ref[...], we_ref[...],
                             preferred_element_type=jnp.float32) + b_ref[...]
    o_ref[...] = jnp.maximum(m, 0.0)


def _messages(g, ea, we, b):
    blk = _EDGE_BLK
    return pl.pallas_call(
        _msg_kernel,
        grid=(_E // blk,),
        in_specs=[pl.BlockSpec((blk, _DP), lambda i: (i, 0)),
                  pl.BlockSpec((blk, _DE), lambda i: (i, 0)),
                  pl.BlockSpec((_DE, _DP), lambda i: (0, 0)),
                  pl.BlockSpec((1, _DP), lambda i: (0, 0))],
        out_specs=pl.BlockSpec((blk, _DP), lambda i: (i, 0)),
        out_shape=jax.ShapeDtypeStruct((_E, _DP), jnp.float32),
    )(g, ea, we, b)


def _update_kernel(agg_ref, h_ref, w2_ref, ws_ref, b_ref, wn_ref,
                   o_ref, hw_ref):
    hn = (jnp.dot(agg_ref[...], w2_ref[...],
                  preferred_element_type=jnp.float32)
          + jnp.dot(h_ref[...], ws_ref[...],
                    preferred_element_type=jnp.float32)
          + b_ref[...])
    hn = jnp.maximum(hn, 0.0)
    o_ref[...] = hn
    hw_ref[...] = jnp.dot(hn, wn_ref[...],
                          preferred_element_type=jnp.float32)


def _update(agg, h, w2, ws, b, wnext):
    blk = _NODE_BLK
    return pl.pallas_call(
        _update_kernel,
        grid=(_N // blk,),
        in_specs=[pl.BlockSpec((blk, _DP), lambda i: (i, 0)),
                  pl.BlockSpec((blk, _DP), lambda i: (i, 0)),
                  pl.BlockSpec((_DP, _DP), lambda i: (0, 0)),
                  pl.BlockSpec((_DP, _DP), lambda i: (0, 0)),
                  pl.BlockSpec((1, _DP), lambda i: (0, 0)),
                  pl.BlockSpec((_DP, _DP), lambda i: (0, 0))],
        out_specs=[pl.BlockSpec((blk, _DP), lambda i: (i, 0)),
                   pl.BlockSpec((blk, _DP), lambda i: (i, 0))],
        out_shape=[jax.ShapeDtypeStruct((_N, _DP), jnp.float32),
                   jax.ShapeDtypeStruct((_N, _DP), jnp.float32)],
    )(agg, h, w2, ws, b, wnext)


def _dot_kernel(a_ref, b_ref, o_ref):
    o_ref[...] = jnp.sum(a_ref[...] * b_ref[...], axis=1, keepdims=True) * 0.1


def _edge_dots(hs, hd):
    blk = _EDGE_BLK
    return pl.pallas_call(
        _dot_kernel,
        grid=(_E // blk,),
        in_specs=[pl.BlockSpec((blk, _DP), lambda i: (i, 0)),
                  pl.BlockSpec((blk, _DP), lambda i: (i, 0))],
        out_specs=pl.BlockSpec((blk, 1), lambda i: (i, 0)),
        out_shape=jax.ShapeDtypeStruct((_E, 1), jnp.float32),
    )(hs, hd)


def _exp_kernel(s_ref, m_ref, o_ref):
    o_ref[...] = jnp.exp(s_ref[...] - m_ref[...])


def _div_kernel(e_ref, d_ref, o_ref):
    o_ref[...] = e_ref[...] / d_ref[...]


def _flat_ew(body, a, b):
    rows = _E // 128
    return pl.pallas_call(
        body,
        out_shape=jax.ShapeDtypeStruct((rows, 128), jnp.float32),
    )(a, b)


_GC = 200


def _sc_row_gather(table, idx):
    info = plsc.get_sparse_core_info()
    nc, ns = info.num_cores, info.num_subcores
    nw = nc * ns
    bpw = _E // nw
    steps = bpw // _GC
    mesh = plsc.VectorSubcoreMesh(core_axis_name="c", subcore_axis_name="s")

    @functools.partial(
        pl.kernel, mesh=mesh,
        out_type=jax.ShapeDtypeStruct((_E, _DP), jnp.float32),
        scratch_types=[
            pltpu.VMEM((bpw,), jnp.int32),
            pltpu.VMEM((_GC, _DP), jnp.float32),
            pltpu.SemaphoreType.DMA,
        ],
    )
    def k(table_hbm, idx_hbm, out_hbm, idx_v, rows_v, sem):
        wid = lax.axis_index("s") * nc + lax.axis_index("c")
        base = wid * bpw
        pltpu.sync_copy(idx_hbm.at[pl.ds(base, bpw)], idx_v)

        def body(j, _):
            off = j * _GC
            pltpu.async_copy(
                table_hbm.at[idx_v.at[pl.ds(off, _GC)]], rows_v, sem).wait()
            pltpu.sync_copy(rows_v, out_hbm.at[pl.ds(base + off, _GC), :])
            return _

        lax.fori_loop(0, steps, body, None)

    return k(table, idx)


def kernel(x, edge_index, edge_attributes, W1, We, b1, W2, Wself, b2):
    src = edge_index[0]
    dst = edge_index[1]

    pad = _DP - _D
    xp = jnp.pad(x, ((0, 0), (0, pad)))
    W1p = jnp.pad(W1, ((0, 0), (0, pad), (0, pad)))
    Wep = jnp.pad(We, ((0, 0), (0, 0), (0, pad)))
    b1p = jnp.pad(b1, ((0, 0), (0, pad))).reshape(_L, 1, _DP)
    W2p = jnp.pad(W2, ((0, 0), (0, pad), (0, pad)))
    Wsp = jnp.pad(Wself, ((0, 0), (0, pad), (0, pad)))
    b2p = jnp.pad(b2, ((0, 0), (0, pad))).reshape(_L, 1, _DP)

    h = xp
    hw = _matmul(h, W1p[0])
    for i in range(_L):
        g = _sc_row_gather(hw, src)
        m = _messages(g, edge_attributes, Wep[i], b1p[i])
        agg = jax.ops.segment_sum(m, dst, num_segments=_N)
        wnext = W1p[i + 1] if i + 1 < _L else W1p[0]
        h, hw = _update(agg, h, W2p[i], Wsp[i], b2p[i], wnext)

    hs = _sc_row_gather(h, src)
    hd = _sc_row_gather(h, dst)
    s = _edge_dots(hs, hd).reshape(_E // 128, 128)

    sf = s.reshape(_E)
    smax = jax.ops.segment_max(sf, src, num_segments=_N)
    smax_g = jnp.take(smax, src, axis=0).reshape(_E // 128, 128)
    ex = _flat_ew(_exp_kernel, s, smax_g)
    denom = jax.ops.segment_sum(ex.reshape(_E), src, num_segments=_N)
    denom_g = jnp.take(denom, src, axis=0).reshape(_E // 128, 128)
    pi = _flat_ew(_div_kernel, ex, denom_g)
    return pi.reshape(_E)


# speedup vs baseline: 1.3893x; 1.0168x over previous
import functools
import jax
import jax.numpy as jnp
import numpy as np
from jax import lax
from jax.experimental import pallas as pl
from jax.experimental.pallas import tpu as pltpu
from jax.experimental.pallas import tpu_sc as plsc

_N = 50000
_E = 800000
_D = 100
_DP = 128
_DE = 16
_L = 4

_NODE_BLK = 1000
_EDGE_BLK = 4000
_FLAT_BLK = 1250


def _mm_kernel(x_ref, w_ref, o_ref):
    o_ref[...] = jnp.dot(x_ref[...], w_ref[...],
                         preferred_element_type=jnp.float32)


def _matmul(x, w):
    n = x.shape[0]
    blk = _NODE_BLK
    return pl.pallas_call(
        _mm_kernel,
        grid=(n // blk,),
        in_specs=[pl.BlockSpec((blk, _DP), lambda i: (i, 0)),
                  pl.BlockSpec((_DP, _DP), lambda i: (0, 0))],
        out_specs=pl.BlockSpec((blk, _DP), lambda i: (i, 0)),
        out_shape=jax.ShapeDtypeStruct((n, _DP), jnp.float32),
    )(x, w)


def _msg_kernel(g_ref, ea_ref, we_ref, b_ref, o_ref):
    m = g_ref[...] + jnp.dot(ea_ref[...], we_ref[...],
                             preferred_element_type=jnp.float32) + b_ref[...]
    o_ref[...] = jnp.maximum(m, 0.0)


def _messages(g, ea, we, b):
    blk = _EDGE_BLK
    return pl.pallas_call(
        _msg_kernel,
        grid=(_E // blk,),
        in_specs=[pl.BlockSpec((blk, _DP), lambda i: (i, 0)),
                  pl.BlockSpec((blk, _DE), lambda i: (i, 0)),
                  pl.BlockSpec((_DE, _DP), lambda i: (0, 0)),
                  pl.BlockSpec((1, _DP), lambda i: (0, 0))],
        out_specs=pl.BlockSpec((blk, _DP), lambda i: (i, 0)),
        out_shape=jax.ShapeDtypeStruct((_E, _DP), jnp.float32),
    )(g, ea, we, b)


def _update_kernel(agg_ref, h_ref, w2_ref, ws_ref, b_ref, wn_ref,
                   o_ref, hw_ref):
    hn = (jnp.dot(agg_ref[...], w2_ref[...],
                  preferred_element_type=jnp.float32)
          + jnp.dot(h_ref[...], ws_ref[...],
                    preferred_element_type=jnp.float32)
          + b_ref[...])
    hn = jnp.maximum(hn, 0.0)
    o_ref[...] = hn
    hw_ref[...] = jnp.dot(hn, wn_ref[...],
                          preferred_element_type=jnp.float32)


def _update(agg, h, w2, ws, b, wnext):
    blk = _NODE_BLK
    return pl.pallas_call(
        _update_kernel,
        grid=(_N // blk,),
        in_specs=[pl.BlockSpec((blk, _DP), lambda i: (i, 0)),
                  pl.BlockSpec((blk, _DP), lambda i: (i, 0)),
                  pl.BlockSpec((_DP, _DP), lambda i: (0, 0)),
                  pl.BlockSpec((_DP, _DP), lambda i: (0, 0)),
                  pl.BlockSpec((1, _DP), lambda i: (0, 0)),
                  pl.BlockSpec((_DP, _DP), lambda i: (0, 0))],
        out_specs=[pl.BlockSpec((blk, _DP), lambda i: (i, 0)),
                   pl.BlockSpec((blk, _DP), lambda i: (i, 0))],
        out_shape=[jax.ShapeDtypeStruct((_N, _DP), jnp.float32),
                   jax.ShapeDtypeStruct((_N, _DP), jnp.float32)],
    )(agg, h, w2, ws, b, wnext)


def _dot_kernel(a_ref, b_ref, o_ref):
    o_ref[...] = jnp.sum(a_ref[...] * b_ref[...], axis=1, keepdims=True) * 0.1


def _edge_dots(hs, hd):
    blk = _EDGE_BLK
    return pl.pallas_call(
        _dot_kernel,
        grid=(_E // blk,),
        in_specs=[pl.BlockSpec((blk, _DP), lambda i: (i, 0)),
                  pl.BlockSpec((blk, _DP), lambda i: (i, 0))],
        out_specs=pl.BlockSpec((blk, 1), lambda i: (i, 0)),
        out_shape=jax.ShapeDtypeStruct((_E, 1), jnp.float32),
    )(hs, hd)


def _exp_kernel(s_ref, m_ref, o_ref):
    o_ref[...] = jnp.exp(s_ref[...] - m_ref[...])


def _div_kernel(e_ref, d_ref, o_ref):
    o_ref[...] = e_ref[...] / d_ref[...]


def _flat_ew(body, a, b):
    rows = _E // 128
    return pl.pallas_call(
        body,
        out_shape=jax.ShapeDtypeStruct((rows, 128), jnp.float32),
    )(a, b)


_GC = 1000


def _sc_row_gather(table, idx):
    info = plsc.get_sparse_core_info()
    nc, ns = info.num_cores, info.num_subcores
    nw = nc * ns
    bpw = _E // nw
    steps = bpw // _GC
    mesh = plsc.VectorSubcoreMesh(core_axis_name="c", subcore_axis_name="s")

    @functools.partial(
        pl.kernel, mesh=mesh,
        out_type=jax.ShapeDtypeStruct((_E, _DP), jnp.float32),
        scratch_types=[
            pltpu.VMEM((_GC,), jnp.int32),
            pltpu.VMEM((_GC, _DP), jnp.float32),
            pltpu.SemaphoreType.DMA,
        ],
    )
    def k(table_hbm, idx_hbm, out_hbm, idx_v, rows_v, sem):
        wid = lax.axis_index("s") * nc + lax.axis_index("c")
        base = wid * bpw

        def body(j, _):
            off = base + j * _GC
            pltpu.sync_copy(idx_hbm.at[pl.ds(off, _GC)], idx_v)
            pltpu.async_copy(table_hbm.at[idx_v], rows_v, sem).wait()
            pltpu.sync_copy(rows_v, out_hbm.at[pl.ds(off, _GC), :])
            return _

        lax.fori_loop(0, steps, body, None)

    return k(table, idx)


def kernel(x, edge_index, edge_attributes, W1, We, b1, W2, Wself, b2):
    src = edge_index[0]
    dst = edge_index[1]

    pad = _DP - _D
    xp = jnp.pad(x, ((0, 0), (0, pad)))
    W1p = jnp.pad(W1, ((0, 0), (0, pad), (0, pad)))
    Wep = jnp.pad(We, ((0, 0), (0, 0), (0, pad)))
    b1p = jnp.pad(b1, ((0, 0), (0, pad))).reshape(_L, 1, _DP)
    W2p = jnp.pad(W2, ((0, 0), (0, pad), (0, pad)))
    Wsp = jnp.pad(Wself, ((0, 0), (0, pad), (0, pad)))
    b2p = jnp.pad(b2, ((0, 0), (0, pad))).reshape(_L, 1, _DP)

    h = xp
    hw = _matmul(h, W1p[0])
    for i in range(_L):
        g = _sc_row_gather(hw, src)
        m = _messages(g, edge_attributes, Wep[i], b1p[i])
        agg = jax.ops.segment_sum(m, dst, num_segments=_N)
        wnext = W1p[i + 1] if i + 1 < _L else W1p[0]
        h, hw = _update(agg, h, W2p[i], Wsp[i], b2p[i], wnext)

    hs = _sc_row_gather(h, src)
    hd = _sc_row_gather(h, dst)
    s = _edge_dots(hs, hd).reshape(_E // 128, 128)

    sf = s.reshape(_E)
    smax = jax.ops.segment_max(sf, src, num_segments=_N)
    smax_g = jnp.take(smax, src, axis=0).reshape(_E // 128, 128)
    ex = _flat_ew(_exp_kernel, s, smax_g)
    denom = jax.ops.segment_sum(ex.reshape(_E), src, num_segments=_N)
    denom_g = jnp.take(denom, src, axis=0).reshape(_E // 128, 128)
    pi = _flat_ew(_div_kernel, ex, denom_g)
    return pi.reshape(_E)


# speedup vs baseline: 1.3907x; 1.0011x over previous
import functools
import jax
import jax.numpy as jnp
import numpy as np
from jax import lax
from jax.experimental import pallas as pl
from jax.experimental.pallas import tpu as pltpu
from jax.experimental.pallas import tpu_sc as plsc

_N = 50000
_E = 800000
_D = 100
_DP = 128
_DE = 16
_L = 4

_NODE_BLK = 1000
_EDGE_BLK = 4000
_FLAT_BLK = 1250


def _mm_kernel(x_ref, w_ref, o_ref):
    o_ref[...] = jnp.dot(x_ref[...], w_ref[...],
                         preferred_element_type=jnp.float32)


def _matmul(x, w):
    n = x.shape[0]
    blk = _NODE_BLK
    return pl.pallas_call(
        _mm_kernel,
        grid=(n // blk,),
        in_specs=[pl.BlockSpec((blk, _DP), lambda i: (i, 0)),
                  pl.BlockSpec((_DP, _DP), lambda i: (0, 0))],
        out_specs=pl.BlockSpec((blk, _DP), lambda i: (i, 0)),
        out_shape=jax.ShapeDtypeStruct((n, _DP), jnp.float32),
    )(x, w)


def _msg_kernel(g_ref, ea_ref, we_ref, b_ref, o_ref):
    m = g_ref[...] + jnp.dot(ea_ref[...], we_ref[...],
                             preferred_element_type=jnp.float32) + b_ref[...]
    o_ref[...] = jnp.maximum(m, 0.0)


def _messages(g, ea, we, b):
    blk = _EDGE_BLK
    return pl.pallas_call(
        _msg_kernel,
        grid=(_E // blk,),
        in_specs=[pl.BlockSpec((blk, _DP), lambda i: (i, 0)),
                  pl.BlockSpec((blk, _DE), lambda i: (i, 0)),
                  pl.BlockSpec((_DE, _DP), lambda i: (0, 0)),
                  pl.BlockSpec((1, _DP), lambda i: (0, 0))],
        out_specs=pl.BlockSpec((blk, _DP), lambda i: (i, 0)),
        out_shape=jax.ShapeDtypeStruct((_E, _DP), jnp.float32),
    )(g, ea, we, b)


def _update_kernel(agg_ref, h_ref, w2_ref, ws_ref, b_ref, wn_ref,
                   o_ref, hw_ref):
    hn = (jnp.dot(agg_ref[...], w2_ref[...],
                  preferred_element_type=jnp.float32)
          + jnp.dot(h_ref[...], ws_ref[...],
                    preferred_element_type=jnp.float32)
          + b_ref[...])
    hn = jnp.maximum(hn, 0.0)
    o_ref[...] = hn
    hw_ref[...] = jnp.dot(hn, wn_ref[...],
                          preferred_element_type=jnp.float32)


def _update(agg, h, w2, ws, b, wnext):
    blk = _NODE_BLK
    return pl.pallas_call(
        _update_kernel,
        grid=(_N // blk,),
        in_specs=[pl.BlockSpec((blk, _DP), lambda i: (i, 0)),
                  pl.BlockSpec((blk, _DP), lambda i: (i, 0)),
                  pl.BlockSpec((_DP, _DP), lambda i: (0, 0)),
                  pl.BlockSpec((_DP, _DP), lambda i: (0, 0)),
                  pl.BlockSpec((1, _DP), lambda i: (0, 0)),
                  pl.BlockSpec((_DP, _DP), lambda i: (0, 0))],
        out_specs=[pl.BlockSpec((blk, _DP), lambda i: (i, 0)),
                   pl.BlockSpec((blk, _DP), lambda i: (i, 0))],
        out_shape=[jax.ShapeDtypeStruct((_N, _DP), jnp.float32),
                   jax.ShapeDtypeStruct((_N, _DP), jnp.float32)],
    )(agg, h, w2, ws, b, wnext)


def _dot_kernel(a_ref, b_ref, o_ref):
    o_ref[...] = jnp.sum(a_ref[...] * b_ref[...], axis=1, keepdims=True) * 0.1


def _edge_dots(hs, hd):
    blk = _EDGE_BLK
    return pl.pallas_call(
        _dot_kernel,
        grid=(_E // blk,),
        in_specs=[pl.BlockSpec((blk, _DP), lambda i: (i, 0)),
                  pl.BlockSpec((blk, _DP), lambda i: (i, 0))],
        out_specs=pl.BlockSpec((blk, 1), lambda i: (i, 0)),
        out_shape=jax.ShapeDtypeStruct((_E, 1), jnp.float32),
    )(hs, hd)


def _exp_kernel(s_ref, m_ref, o_ref):
    o_ref[...] = jnp.exp(s_ref[...] - m_ref[...])


def _div_kernel(e_ref, d_ref, o_ref):
    o_ref[...] = e_ref[...] / d_ref[...]


def _flat_ew(body, a, b):
    rows = _E // 128
    return pl.pallas_call(
        body,
        out_shape=jax.ShapeDtypeStruct((rows, 128), jnp.float32),
    )(a, b)


_GC = 200


def _sc_row_gather(table, idx):
    info = plsc.get_sparse_core_info()
    nc, ns = info.num_cores, info.num_subcores
    nw = nc * ns
    bpw = _E // nw
    steps = bpw // _GC
    mesh = plsc.VectorSubcoreMesh(core_axis_name="c", subcore_axis_name="s")

    @functools.partial(
        pl.kernel, mesh=mesh,
        out_type=jax.ShapeDtypeStruct((_E, _DP), jnp.float32),
        scratch_types=[
            pltpu.VMEM((bpw,), jnp.int32),
            pltpu.VMEM((_GC, _DP), jnp.float32),
            pltpu.VMEM((_GC, _DP), jnp.float32),
            pltpu.SemaphoreType.DMA,
            pltpu.SemaphoreType.DMA,
        ],
    )
    def k(table_hbm, idx_hbm, out_hbm, idx_v, rows_a, rows_b, sem_a, sem_b):
        wid = lax.axis_index("s") * nc + lax.axis_index("c")
        base = wid * bpw
        pltpu.sync_copy(idx_hbm.at[pl.ds(base, bpw)], idx_v)
        rows = (rows_a, rows_b)
        sems = (sem_a, sem_b)
        cps = [None, None]
        for j in range(steps):
            b = j & 1
            cps[b] = pltpu.async_copy(
                table_hbm.at[idx_v.at[pl.ds(j * _GC, _GC)]], rows[b], sems[b])
            if j >= 1:
                pb = 1 - b
                cps[pb].wait()
                pltpu.sync_copy(
                    rows[pb], out_hbm.at[pl.ds(base + (j - 1) * _GC, _GC), :])
        last = steps - 1
        cps[last & 1].wait()
        pltpu.sync_copy(
            rows[last & 1], out_hbm.at[pl.ds(base + last * _GC, _GC), :])

    return k(table, idx)


def kernel(x, edge_index, edge_attributes, W1, We, b1, W2, Wself, b2):
    src = edge_index[0]
    dst = edge_index[1]

    pad = _DP - _D
    xp = jnp.pad(x, ((0, 0), (0, pad)))
    W1p = jnp.pad(W1, ((0, 0), (0, pad), (0, pad)))
    Wep = jnp.pad(We, ((0, 0), (0, 0), (0, pad)))
    b1p = jnp.pad(b1, ((0, 0), (0, pad))).reshape(_L, 1, _DP)
    W2p = jnp.pad(W2, ((0, 0), (0, pad), (0, pad)))
    Wsp = jnp.pad(Wself, ((0, 0), (0, pad), (0, pad)))
    b2p = jnp.pad(b2, ((0, 0), (0, pad))).reshape(_L, 1, _DP)

    h = xp
    hw = _matmul(h, W1p[0])
    for i in range(_L):
        g = _sc_row_gather(hw, src)
        m = _messages(g, edge_attributes, Wep[i], b1p[i])
        agg = jax.ops.segment_sum(m, dst, num_segments=_N)
        wnext = W1p[i + 1] if i + 1 < _L else W1p[0]
        h, hw = _update(agg, h, W2p[i], Wsp[i], b2p[i], wnext)

    hs = _sc_row_gather(h, src)
    hd = _sc_row_gather(h, dst)
    s = _edge_dots(hs, hd).reshape(_E // 128, 128)

    sf = s.reshape(_E)
    smax = jax.ops.segment_max(sf, src, num_segments=_N)
    smax_g = jnp.take(smax, src, axis=0).reshape(_E // 128, 128)
    ex = _flat_ew(_exp_kernel, s, smax_g)
    denom = jax.ops.segment_sum(ex.reshape(_E), src, num_segments=_N)
    denom_g = jnp.take(denom, src, axis=0).reshape(_E // 128, 128)
    pi = _flat_ew(_div_kernel, ex, denom_g)
    return pi.reshape(_E)
